# Initial kernel scaffold; baseline (speedup 1.0000x reference)
import jax
import jax.numpy as jnp
from jax.experimental import pallas as pl


def kernel(inputs):
    raise NotImplementedError("write your pallas kernel here")


# speedup vs baseline: 9.2550x; 9.2550x over previous
import functools

import jax
import jax.numpy as jnp
from jax.experimental import pallas as pl

_K = 64


def _body(x_ref, o_ref):
    mask = jnp.int32(0x7FFFFFFF)
    x = x_ref[...]
    rows = x.shape[0]
    b = jax.lax.bitcast_convert_type(x, jnp.int32)
    keys = jnp.where(b < 0, b ^ mask, b)
    xmax = jnp.max(x, axis=1, keepdims=True)

    def step(i, t):
        bit = jax.lax.shift_left(jnp.int32(1), jnp.int32(31) - i)
        cand = t ^ bit
        cnt = jnp.sum((keys >= cand).astype(jnp.int32), axis=1, keepdims=True)
        return jnp.where(cnt >= _K, cand, t)

    t0 = jnp.full((rows, 1), jnp.iinfo(jnp.int32).min, dtype=jnp.int32)
    t = jax.lax.fori_loop(0, 32, step, t0, unroll=True)

    tf = jax.lax.bitcast_convert_type(jnp.where(t < 0, t ^ mask, t), jnp.float32)

    p = jnp.exp(x - xmax)
    gt = keys > t
    ge = keys >= t
    c_gt = jnp.sum(gt.astype(jnp.float32), axis=1, keepdims=True)
    sum_gt = jnp.sum(jnp.where(gt, p, 0.0), axis=1, keepdims=True)
    denom = sum_gt + (jnp.float32(_K) - c_gt) * jnp.exp(tf - xmax)
    o_ref[...] = jnp.where(ge, p / denom, 0.0)


@functools.partial(jax.jit, static_argnums=())
def kernel(inputs):
    n_rows, depth = inputs.shape
    block_rows = 16
    grid = (n_rows // block_rows,)
    return pl.pallas_call(
        _body,
        grid=grid,
        in_specs=[pl.BlockSpec((block_rows, depth), lambda i: (i, 0))],
        out_specs=pl.BlockSpec((block_rows, depth), lambda i: (i, 0)),
        out_shape=jax.ShapeDtypeStruct((n_rows, depth), jnp.float32),
    )(inputs)


# speedup vs baseline: 13.8017x; 1.4913x over previous
import functools

import jax
import jax.numpy as jnp
from jax.experimental import pallas as pl

_K = 64


def _count16(pred):
    def rec(lo, hi):
        if hi - lo == 128:
            return jnp.where(pred(lo, hi), jnp.int16(1), jnp.int16(0))
        mid = (lo + hi) // 2
        return rec(lo, mid) + rec(mid, hi)

    m = rec(0, 32768)
    return jnp.sum(m.astype(jnp.int32), axis=1, keepdims=True)


def _as_i16(pat):
    return (pat - 32768).astype(jnp.int16)


def _bisect16(h, rows, extra=None):
    def step(i, t):
        bit = jax.lax.shift_left(jnp.int32(1), jnp.int32(15) - i)
        cand = t | bit
        c16 = _as_i16(cand)
        cnt = _count16(lambda lo, hi: h[:, lo:hi] >= c16)
        if extra is not None:
            cnt = cnt + extra
        return jnp.where(cnt >= _K, cand, t)

    t0 = jnp.zeros((rows, 1), dtype=jnp.int32)
    return jax.lax.fori_loop(0, 16, step, t0, unroll=True)


def _body(x_ref, o_ref):
    mask = jnp.int32(0x7FFFFFFF)
    x = x_ref[...]
    rows = x.shape[0]
    b = jax.lax.bitcast_convert_type(x, jnp.int32)
    keys = jnp.where(b < 0, b ^ mask, b)
    xmax = jnp.max(x, axis=1, keepdims=True)

    hi = jax.lax.shift_right_arithmetic(keys, 16).astype(jnp.int16)
    p_pat = _bisect16(hi, rows)
    p16 = _as_i16(p_pat)

    c_hi_gt = _count16(lambda lo, hi_: hi[:, lo:hi_] > p16)
    lo = _as_i16(keys & jnp.int32(0xFFFF))
    lo_m = jnp.where(hi == p16, lo, jnp.int16(-32768))
    tl_pat = _bisect16(lo_m, rows, extra=c_hi_gt)

    t = jax.lax.shift_left(p_pat - 32768, 16) | tl_pat

    tf = jax.lax.bitcast_convert_type(jnp.where(t < 0, t ^ mask, t), jnp.float32)

    e = jnp.exp(x - xmax)
    gt = keys > t
    ge = keys >= t
    c_gt = jnp.sum(gt.astype(jnp.float32), axis=1, keepdims=True)
    sum_gt = jnp.sum(jnp.where(gt, e, 0.0), axis=1, keepdims=True)
    denom = sum_gt + (jnp.float32(_K) - c_gt) * jnp.exp(tf - xmax)
    o_ref[...] = jnp.where(ge, e / denom, 0.0)


@functools.partial(jax.jit, static_argnums=())
def kernel(inputs):
    n_rows, depth = inputs.shape
    block_rows = 16
    grid = (n_rows // block_rows,)
    return pl.pallas_call(
        _body,
        grid=grid,
        in_specs=[pl.BlockSpec((block_rows, depth), lambda i: (i, 0))],
        out_specs=pl.BlockSpec((block_rows, depth), lambda i: (i, 0)),
        out_shape=jax.ShapeDtypeStruct((n_rows, depth), jnp.float32),
    )(inputs)


# speedup vs baseline: 16.6692x; 1.2078x over previous
import functools

import jax
import jax.numpy as jnp
from jax.experimental import pallas as pl

_K = 64


def _count16(pred):
    def rec(lo, hi):
        if hi - lo == 128:
            return jnp.where(pred(lo, hi), jnp.int16(1), jnp.int16(0))
        mid = (lo + hi) // 2
        return rec(lo, mid) + rec(mid, hi)

    m = rec(0, 32768)
    return jnp.sum(m.astype(jnp.int32), axis=1, keepdims=True)


def _as_i16(pat):
    return (pat - 32768).astype(jnp.int16)


def _bisect16(h, rows, extra=None):
    def step(i, t):
        bit = jax.lax.shift_left(jnp.int32(1), jnp.int32(15) - i)
        cand = t | bit
        c16 = _as_i16(cand)
        cnt = _count16(lambda lo, hi: h[:, lo:hi] >= c16)
        if extra is not None:
            cnt = cnt + extra
        return jnp.where(cnt >= _K, cand, t)

    t0 = jnp.zeros((rows, 1), dtype=jnp.int32)
    return jax.lax.fori_loop(0, 16, step, t0, unroll=True)


def _body(x_ref, o_ref):
    mask = jnp.int32(0x7FFFFFFF)
    x = x_ref[...]
    rows = x.shape[0]
    b = jax.lax.bitcast_convert_type(x, jnp.int32)
    keys = jnp.where(b < 0, b ^ mask, b)
    xmax = jnp.max(x, axis=1, keepdims=True)

    hi = jax.lax.shift_right_arithmetic(keys, 16).astype(jnp.int16)
    p_pat = _bisect16(hi, rows)
    p16 = _as_i16(p_pat)

    c_hi_gt = _count16(lambda lo, hi_: hi[:, lo:hi_] > p16)
    lo = _as_i16(keys & jnp.int32(0xFFFF))
    lo_m = jnp.where(hi == p16, lo, jnp.int16(-32768))
    tl_pat = _bisect16(lo_m, rows, extra=c_hi_gt)

    t = jax.lax.shift_left(p_pat - 32768, 16) | tl_pat

    tf = jax.lax.bitcast_convert_type(jnp.where(t < 0, t ^ mask, t), jnp.float32)

    e = jnp.exp(x - xmax)
    ge = keys >= t
    em = jnp.where(ge, e, 0.0)
    c_ge = jnp.sum(ge.astype(jnp.float32), axis=1, keepdims=True)
    sum_ge = jnp.sum(em, axis=1, keepdims=True)
    denom = sum_ge - (c_ge - jnp.float32(_K)) * jnp.exp(tf - xmax)
    o_ref[...] = em / denom


@functools.partial(jax.jit, static_argnums=())
def kernel(inputs):
    n_rows, depth = inputs.shape
    block_rows = 32
    grid = (n_rows // block_rows,)
    return pl.pallas_call(
        _body,
        grid=grid,
        in_specs=[pl.BlockSpec((block_rows, depth), lambda i: (i, 0))],
        out_specs=pl.BlockSpec((block_rows, depth), lambda i: (i, 0)),
        out_shape=jax.ShapeDtypeStruct((n_rows, depth), jnp.float32),
    )(inputs)
